# initial kernel scaffold (unmeasured)
import jax
import jax.numpy as jnp
from jax import lax
from jax.experimental import pallas as pl
from jax.experimental.pallas import tpu as pltpu

N_DEV = 16
M_BLK = 256

_DEV_ID_TYPE = getattr(pltpu, "DeviceIdType", None) or pl.DeviceIdType


def kernel(x, w_mat):
    m_full, k_shard = x.shape
    k_full, n = w_mat.shape
    assert m_full == N_DEV * M_BLK and k_shard == M_BLK

    def body(x_ref, w_ref, out_ref, xg_ref, wbuf_ref, amax_ref,
             ssem, rsem, assem, arsem, wsem):
        my = lax.axis_index("i")

        def j_of(d):
            return lax.rem(my + N_DEV - d, N_DEV)

        send_descs = []
        for d in range(1, N_DEV):
            dst = lax.rem(my + d, N_DEV)
            rdma = pltpu.make_async_remote_copy(
                src_ref=x_ref.at[pl.ds(dst * M_BLK, M_BLK), :],
                dst_ref=xg_ref.at[my],
                send_sem=ssem.at[d - 1],
                recv_sem=rsem.at[d - 1],
                device_id=(dst,),
                device_id_type=_DEV_ID_TYPE.MESH,
            )
            rdma.start()
            send_descs.append(rdma)

        xg_ref[my, :, :] = x_ref[pl.ds(my * M_BLK, M_BLK), :]

        def w_dma(d):
            j = j_of(d)
            return pltpu.make_async_copy(
                w_ref.at[pl.ds(j * M_BLK, M_BLK), :],
                wbuf_ref.at[d % 2],
                wsem.at[d % 2],
            )

        dmas = {}
        for d in range(2):
            dmas[d] = w_dma(d)
            dmas[d].start()

        for d in range(N_DEV):
            if d > 0:
                recv = pltpu.make_async_remote_copy(
                    src_ref=x_ref.at[pl.ds(0, M_BLK), :],
                    dst_ref=xg_ref.at[j_of(d)],
                    send_sem=ssem.at[d - 1],
                    recv_sem=rsem.at[d - 1],
                    device_id=(my,),
                    device_id_type=_DEV_ID_TYPE.MESH,
                )
                recv.wait_recv()
            dmas[d].wait()
            contrib = jnp.dot(
                xg_ref[j_of(d)], wbuf_ref[d % 2],
                preferred_element_type=jnp.float32,
            )
            if d + 2 < N_DEV:
                dmas[d + 2] = w_dma(d + 2)
                dmas[d + 2].start()
            if d == 0:
                out_ref[...] = contrib
            else:
                out_ref[...] += contrib

        for rdma in send_descs:
            rdma.wait_send()

        local_amax = jnp.max(jnp.abs(out_ref[...]))
        amax_ref[pl.ds(my, 1), :] = (
            jnp.zeros((1, 128), jnp.float32) + local_amax
        )
        a_sends = []
        for d in range(1, N_DEV):
            dst = lax.rem(my + d, N_DEV)
            r = pltpu.make_async_remote_copy(
                src_ref=amax_ref.at[pl.ds(my, 1), :],
                dst_ref=amax_ref.at[pl.ds(my, 1), :],
                send_sem=assem.at[d - 1],
                recv_sem=arsem.at[d - 1],
                device_id=(dst,),
                device_id_type=_DEV_ID_TYPE.MESH,
            )
            r.start()
            a_sends.append(r)
        for d in range(1, N_DEV):
            recv = pltpu.make_async_remote_copy(
                src_ref=amax_ref.at[pl.ds(j_of(d), 1), :],
                dst_ref=amax_ref.at[pl.ds(j_of(d), 1), :],
                send_sem=assem.at[d - 1],
                recv_sem=arsem.at[d - 1],
                device_id=(my,),
                device_id_type=_DEV_ID_TYPE.MESH,
            )
            recv.wait_recv()
        for r in a_sends:
            r.wait_send()

        g_amax = jnp.max(amax_ref[...])
        scale = g_amax / 127.0
        q = jnp.clip(jnp.round(out_ref[...] / scale), -127.0, 127.0)
        out_ref[...] = q * scale

    return pl.pallas_call(
        body,
        out_shape=jax.ShapeDtypeStruct((M_BLK, n), jnp.float32),
        in_specs=[
            pl.BlockSpec(memory_space=pltpu.VMEM),
            pl.BlockSpec(memory_space=pltpu.ANY),
        ],
        out_specs=pl.BlockSpec(memory_space=pltpu.VMEM),
        scratch_shapes=[
            pltpu.VMEM((N_DEV, M_BLK, M_BLK), jnp.float32),
            pltpu.VMEM((2, M_BLK, n), jnp.float32),
            pltpu.VMEM((N_DEV, 128), jnp.float32),
            pltpu.SemaphoreType.DMA((N_DEV - 1,)),
            pltpu.SemaphoreType.DMA((N_DEV - 1,)),
            pltpu.SemaphoreType.DMA((N_DEV - 1,)),
            pltpu.SemaphoreType.DMA((N_DEV - 1,)),
            pltpu.SemaphoreType.DMA((2,)),
        ],
    )(x, w_mat)


# baseline (device time: 86670 ns/iter reference)
import jax
import jax.numpy as jnp
from jax import lax
from jax.experimental import pallas as pl
from jax.experimental.pallas import tpu as pltpu

N_DEV = 16
M_BLK = 256

_DEV_ID_TYPE = getattr(pltpu, "DeviceIdType", None) or pl.DeviceIdType


def kernel(x, w_mat):
    m_full, k_shard = x.shape
    k_full, n = w_mat.shape
    assert m_full == N_DEV * M_BLK and k_shard == M_BLK

    def body(x_ref, w_ref, out_ref, xg_ref, wbuf_ref, amax_ref,
             ssem, rsem, assem, arsem, wsem):
        my = lax.axis_index("i")

        def j_of(d):
            return lax.rem(my + N_DEV - d, N_DEV)

        barrier_sem = pltpu.get_barrier_semaphore()
        for d in range(1, N_DEV):
            pl.semaphore_signal(
                barrier_sem, inc=1,
                device_id=(lax.rem(my + d, N_DEV),),
                device_id_type=_DEV_ID_TYPE.MESH,
            )
        pl.semaphore_wait(barrier_sem, N_DEV - 1)

        send_descs = []
        for d in range(1, N_DEV):
            dst = lax.rem(my + d, N_DEV)
            rdma = pltpu.make_async_remote_copy(
                src_ref=x_ref.at[pl.ds(dst * M_BLK, M_BLK), :],
                dst_ref=xg_ref.at[my],
                send_sem=ssem.at[d - 1],
                recv_sem=rsem.at[d - 1],
                device_id=(dst,),
                device_id_type=_DEV_ID_TYPE.MESH,
            )
            rdma.start()
            send_descs.append(rdma)

        xg_ref[my, :, :] = x_ref[pl.ds(my * M_BLK, M_BLK), :]

        def w_dma(d):
            j = j_of(d)
            return pltpu.make_async_copy(
                w_ref.at[pl.ds(j * M_BLK, M_BLK), :],
                wbuf_ref.at[d % 2],
                wsem.at[d % 2],
            )

        dmas = {}
        for d in range(2):
            dmas[d] = w_dma(d)
            dmas[d].start()

        for d in range(N_DEV):
            if d > 0:
                recv = pltpu.make_async_remote_copy(
                    src_ref=x_ref.at[pl.ds(0, M_BLK), :],
                    dst_ref=xg_ref.at[j_of(d)],
                    send_sem=ssem.at[d - 1],
                    recv_sem=rsem.at[d - 1],
                    device_id=(my,),
                    device_id_type=_DEV_ID_TYPE.MESH,
                )
                recv.wait_recv()
            dmas[d].wait()
            contrib = jnp.dot(
                xg_ref[j_of(d)], wbuf_ref[d % 2],
                preferred_element_type=jnp.float32,
            )
            if d + 2 < N_DEV:
                dmas[d + 2] = w_dma(d + 2)
                dmas[d + 2].start()
            if d == 0:
                out_ref[...] = contrib
            else:
                out_ref[...] += contrib

        for rdma in send_descs:
            rdma.wait_send()

        local_amax = jnp.max(jnp.abs(out_ref[...]))
        amax_ref[pl.ds(my, 1), :] = (
            jnp.zeros((1, 128), jnp.float32) + local_amax
        )
        a_sends = []
        for d in range(1, N_DEV):
            dst = lax.rem(my + d, N_DEV)
            r = pltpu.make_async_remote_copy(
                src_ref=amax_ref.at[pl.ds(my, 1), :],
                dst_ref=amax_ref.at[pl.ds(my, 1), :],
                send_sem=assem.at[d - 1],
                recv_sem=arsem.at[d - 1],
                device_id=(dst,),
                device_id_type=_DEV_ID_TYPE.MESH,
            )
            r.start()
            a_sends.append(r)
        for d in range(1, N_DEV):
            recv = pltpu.make_async_remote_copy(
                src_ref=amax_ref.at[pl.ds(j_of(d), 1), :],
                dst_ref=amax_ref.at[pl.ds(j_of(d), 1), :],
                send_sem=assem.at[d - 1],
                recv_sem=arsem.at[d - 1],
                device_id=(my,),
                device_id_type=_DEV_ID_TYPE.MESH,
            )
            recv.wait_recv()
        for r in a_sends:
            r.wait_send()

        g_amax = jnp.max(amax_ref[...])
        scale = g_amax / 127.0
        q = jnp.clip(jnp.round(out_ref[...] / scale), -127.0, 127.0)
        out_ref[...] = q * scale

    return pl.pallas_call(
        body,
        out_shape=jax.ShapeDtypeStruct((M_BLK, n), jnp.float32),
        in_specs=[
            pl.BlockSpec(memory_space=pltpu.VMEM),
            pl.BlockSpec(memory_space=pl.ANY),
        ],
        out_specs=pl.BlockSpec(memory_space=pltpu.VMEM),
        scratch_shapes=[
            pltpu.VMEM((N_DEV, M_BLK, M_BLK), jnp.float32),
            pltpu.VMEM((2, M_BLK, n), jnp.float32),
            pltpu.VMEM((N_DEV, 128), jnp.float32),
            pltpu.SemaphoreType.DMA((N_DEV - 1,)),
            pltpu.SemaphoreType.DMA((N_DEV - 1,)),
            pltpu.SemaphoreType.DMA((N_DEV - 1,)),
            pltpu.SemaphoreType.DMA((N_DEV - 1,)),
            pltpu.SemaphoreType.DMA((2,)),
        ],
        compiler_params=pltpu.CompilerParams(collective_id=0),
    )(x, w_mat)


# device time: 71434 ns/iter; 1.2133x vs baseline; 1.2133x over previous
import os

import jax
import jax.numpy as jnp
from jax import lax
from jax.experimental import pallas as pl
from jax.experimental.pallas import tpu as pltpu

N_DEV = 16
M_BLK = 256

_KVAR = os.environ.get("KVAR", "")

_DEV_ID_TYPE = getattr(pltpu, "DeviceIdType", None) or pl.DeviceIdType


def kernel(x, w_mat):
    m_full, k_shard = x.shape
    k_full, n = w_mat.shape
    assert m_full == N_DEV * M_BLK and k_shard == M_BLK

    def body(x_ref, w_ref, out_ref, xbf_ref, xg_ref, wbuf_ref, amax_ref,
             ssem, rsem, assem, arsem, wsem):
        my = lax.axis_index("i")

        def j_of(d):
            return lax.rem(my + N_DEV - d, N_DEV)

        send_descs = []
        if _KVAR != "nocomm":
            barrier_sem = pltpu.get_barrier_semaphore()
            for d in range(1, N_DEV):
                pl.semaphore_signal(
                    barrier_sem, inc=1,
                    device_id=(lax.rem(my + d, N_DEV),),
                    device_id_type=_DEV_ID_TYPE.MESH,
                )
            pl.semaphore_wait(barrier_sem, N_DEV - 1)

        xbf_ref[...] = x_ref[...].astype(jnp.bfloat16)

        if _KVAR != "nocomm":
            for d in range(1, N_DEV):
                dst = lax.rem(my + d, N_DEV)
                rdma = pltpu.make_async_remote_copy(
                    src_ref=xbf_ref.at[pl.ds(dst * M_BLK, M_BLK), :],
                    dst_ref=xg_ref.at[my],
                    send_sem=ssem.at[d - 1],
                    recv_sem=rsem.at[d - 1],
                    device_id=(dst,),
                    device_id_type=_DEV_ID_TYPE.MESH,
                )
                rdma.start()
                send_descs.append(rdma)

        xg_ref[my, :, :] = xbf_ref[pl.ds(my * M_BLK, M_BLK), :]

        def w_dma(d):
            j = j_of(d)
            return pltpu.make_async_copy(
                w_ref.at[pl.ds(j * M_BLK, M_BLK), :],
                wbuf_ref.at[d % 2],
                wsem.at[d % 2],
            )

        dmas = {}
        for d in range(2):
            dmas[d] = w_dma(d)
            dmas[d].start()

        for d in range(N_DEV):
            if d > 0 and _KVAR != "nocomm":
                recv = pltpu.make_async_remote_copy(
                    src_ref=xbf_ref.at[pl.ds(0, M_BLK), :],
                    dst_ref=xg_ref.at[j_of(d)],
                    send_sem=ssem.at[d - 1],
                    recv_sem=rsem.at[d - 1],
                    device_id=(my,),
                    device_id_type=_DEV_ID_TYPE.MESH,
                )
                recv.wait_recv()
            dmas[d].wait()
            contrib = jnp.dot(
                xg_ref[j_of(d)].astype(jnp.float32), wbuf_ref[d % 2],
                preferred_element_type=jnp.float32,
            )
            if d + 2 < N_DEV:
                dmas[d + 2] = w_dma(d + 2)
                dmas[d + 2].start()
            if d == 0:
                out_ref[...] = contrib
            else:
                out_ref[...] += contrib

        for rdma in send_descs:
            rdma.wait_send()

        local_amax = jnp.max(jnp.abs(out_ref[...]))
        if _KVAR != "nocomm":
            amax_ref[pl.ds(my, 1), :] = (
                jnp.zeros((1, 128), jnp.float32) + local_amax
            )
            a_sends = []
            for d in range(1, N_DEV):
                dst = lax.rem(my + d, N_DEV)
                r = pltpu.make_async_remote_copy(
                    src_ref=amax_ref.at[pl.ds(my, 1), :],
                    dst_ref=amax_ref.at[pl.ds(my, 1), :],
                    send_sem=assem.at[d - 1],
                    recv_sem=arsem.at[d - 1],
                    device_id=(dst,),
                    device_id_type=_DEV_ID_TYPE.MESH,
                )
                r.start()
                a_sends.append(r)
            for d in range(1, N_DEV):
                recv = pltpu.make_async_remote_copy(
                    src_ref=amax_ref.at[pl.ds(j_of(d), 1), :],
                    dst_ref=amax_ref.at[pl.ds(j_of(d), 1), :],
                    send_sem=assem.at[d - 1],
                    recv_sem=arsem.at[d - 1],
                    device_id=(my,),
                    device_id_type=_DEV_ID_TYPE.MESH,
                )
                recv.wait_recv()
            for r in a_sends:
                r.wait_send()
            g_amax = jnp.max(amax_ref[...])
        else:
            g_amax = local_amax

        if _KVAR != "noquant":
            inv_scale = 127.0 / g_amax
            scale = g_amax / 127.0
            q = jnp.clip(
                jnp.round(out_ref[...] * inv_scale), -127.0, 127.0
            )
            out_ref[...] = q * scale

    return pl.pallas_call(
        body,
        out_shape=jax.ShapeDtypeStruct((M_BLK, n), jnp.float32),
        in_specs=[
            pl.BlockSpec(memory_space=pltpu.VMEM),
            pl.BlockSpec(memory_space=pl.ANY),
        ],
        out_specs=pl.BlockSpec(memory_space=pltpu.VMEM),
        scratch_shapes=[
            pltpu.VMEM((N_DEV * M_BLK, M_BLK), jnp.bfloat16),
            pltpu.VMEM((N_DEV, M_BLK, M_BLK), jnp.bfloat16),
            pltpu.VMEM((2, M_BLK, n), jnp.float32),
            pltpu.VMEM((N_DEV, 128), jnp.float32),
            pltpu.SemaphoreType.DMA((N_DEV - 1,)),
            pltpu.SemaphoreType.DMA((N_DEV - 1,)),
            pltpu.SemaphoreType.DMA((N_DEV - 1,)),
            pltpu.SemaphoreType.DMA((N_DEV - 1,)),
            pltpu.SemaphoreType.DMA((2,)),
        ],
        compiler_params=pltpu.CompilerParams(
            collective_id=None if _KVAR == "nocomm" else 0,
            vmem_limit_bytes=100 * 1024 * 1024,
        ),
    )(x, w_mat)


# device time: 66748 ns/iter; 1.2985x vs baseline; 1.0702x over previous
import os

import jax
import jax.numpy as jnp
from jax import lax
from jax.experimental import pallas as pl
from jax.experimental.pallas import tpu as pltpu

N_DEV = 16
M_BLK = 256

_FLAGS = set(filter(None, os.environ.get("KVAR", "").split(",")))
_NOCOMM = "nocomm" in _FLAGS
_NOQUANT = "noquant" in _FLAGS
_BF16W = "bf16w" in _FLAGS
_DMAONLY = "dmaonly" in _FLAGS
_NBUF = int(os.environ.get("NBUF", "2"))
_WSPLIT = int(os.environ.get("WSPLIT", "1"))

_DEV_ID_TYPE = getattr(pltpu, "DeviceIdType", None) or pl.DeviceIdType


def kernel(x, w_mat):
    m_full, k_shard = x.shape
    k_full, n = w_mat.shape
    assert m_full == N_DEV * M_BLK and k_shard == M_BLK

    def body(x_ref, w_ref, out_ref, xbf_ref, xg_ref, wbuf_ref, wbf_ref,
             amax_ref, ssem, rsem, assem, arsem, wsem):
        my = lax.axis_index("i")

        def j_of(d):
            return lax.rem(my + N_DEV - d, N_DEV)

        order = [0]
        for k in range(1, N_DEV // 2 + 1):
            order.append(k)
            if k != N_DEV - k:
                order.append(N_DEV - k)

        send_descs = []
        if not _NOCOMM:
            barrier_sem = pltpu.get_barrier_semaphore()
            for d in range(1, N_DEV):
                pl.semaphore_signal(
                    barrier_sem, inc=1,
                    device_id=(lax.rem(my + d, N_DEV),),
                    device_id_type=_DEV_ID_TYPE.MESH,
                )
            pl.semaphore_wait(barrier_sem, N_DEV - 1)

        xbf_ref[...] = x_ref[...].astype(jnp.bfloat16)

        if not _NOCOMM:
            for d in order[1:]:
                dst = lax.rem(my + d, N_DEV)
                rdma = pltpu.make_async_remote_copy(
                    src_ref=xbf_ref.at[pl.ds(dst * M_BLK, M_BLK), :],
                    dst_ref=xg_ref.at[my],
                    send_sem=ssem.at[d - 1],
                    recv_sem=rsem.at[d - 1],
                    device_id=(dst,),
                    device_id_type=_DEV_ID_TYPE.MESH,
                )
                rdma.start()
                send_descs.append(rdma)

        xg_ref[my, :, :] = xbf_ref[pl.ds(my * M_BLK, M_BLK), :]

        rows_per = M_BLK // _WSPLIT

        def w_dma(d, slot):
            j = j_of(d)
            descs = []
            for h in range(_WSPLIT):
                descs.append(pltpu.make_async_copy(
                    w_ref.at[pl.ds(j * M_BLK + h * rows_per, rows_per), :],
                    wbuf_ref.at[slot, pl.ds(h * rows_per, rows_per), :],
                    wsem.at[slot, h],
                ))
            return descs

        def start_all(descs):
            for c in descs:
                c.start()

        def wait_all(descs):
            for c in descs:
                c.wait()

        dmas = {}
        for idx in range(_NBUF):
            dmas[idx] = w_dma(order[idx], idx % _NBUF)
            start_all(dmas[idx])

        for idx in range(N_DEV):
            d = order[idx]
            if d > 0 and not _NOCOMM:
                recv = pltpu.make_async_remote_copy(
                    src_ref=xbf_ref.at[pl.ds(0, M_BLK), :],
                    dst_ref=xg_ref.at[j_of(d)],
                    send_sem=ssem.at[d - 1],
                    recv_sem=rsem.at[d - 1],
                    device_id=(my,),
                    device_id_type=_DEV_ID_TYPE.MESH,
                )
                recv.wait_recv()
            wait_all(dmas[idx])
            slot = idx % _NBUF
            if _DMAONLY:
                if idx + _NBUF < N_DEV:
                    dmas[idx + _NBUF] = w_dma(order[idx + _NBUF], slot)
                    start_all(dmas[idx + _NBUF])
                if idx == N_DEV - 1:
                    out_ref[...] = wbuf_ref[slot]
                continue
            if _BF16W:
                wbf_ref[idx % 2] = wbuf_ref[slot].astype(jnp.bfloat16)
                contrib = jnp.dot(
                    xg_ref[j_of(d)], wbf_ref[idx % 2],
                    preferred_element_type=jnp.float32,
                )
            else:
                contrib = jnp.dot(
                    xg_ref[j_of(d)].astype(jnp.float32), wbuf_ref[slot],
                    preferred_element_type=jnp.float32,
                )
            if idx + _NBUF < N_DEV:
                dmas[idx + _NBUF] = w_dma(order[idx + _NBUF], slot)
                start_all(dmas[idx + _NBUF])
            if idx == 0:
                out_ref[...] = contrib
            else:
                out_ref[...] += contrib

        for rdma in send_descs:
            rdma.wait_send()

        local_amax = jnp.max(jnp.abs(out_ref[...]))
        if not _NOCOMM:
            amax_ref[pl.ds(my, 1), :] = (
                jnp.zeros((1, 128), jnp.float32) + local_amax
            )
            a_sends = []
            for d in range(1, N_DEV):
                dst = lax.rem(my + d, N_DEV)
                r = pltpu.make_async_remote_copy(
                    src_ref=amax_ref.at[pl.ds(my, 1), :],
                    dst_ref=amax_ref.at[pl.ds(my, 1), :],
                    send_sem=assem.at[d - 1],
                    recv_sem=arsem.at[d - 1],
                    device_id=(dst,),
                    device_id_type=_DEV_ID_TYPE.MESH,
                )
                r.start()
                a_sends.append(r)
            for d in range(1, N_DEV):
                recv = pltpu.make_async_remote_copy(
                    src_ref=amax_ref.at[pl.ds(j_of(d), 1), :],
                    dst_ref=amax_ref.at[pl.ds(j_of(d), 1), :],
                    send_sem=assem.at[d - 1],
                    recv_sem=arsem.at[d - 1],
                    device_id=(my,),
                    device_id_type=_DEV_ID_TYPE.MESH,
                )
                recv.wait_recv()
            for r in a_sends:
                r.wait_send()
            g_amax = jnp.max(amax_ref[...])
        else:
            g_amax = local_amax

        if not _NOQUANT:
            inv_scale = 127.0 / g_amax
            scale = g_amax / 127.0
            q = jnp.clip(
                jnp.round(out_ref[...] * inv_scale), -127.0, 127.0
            )
            out_ref[...] = q * scale

    return pl.pallas_call(
        body,
        out_shape=jax.ShapeDtypeStruct((M_BLK, n), jnp.float32),
        in_specs=[
            pl.BlockSpec(memory_space=pltpu.VMEM),
            pl.BlockSpec(memory_space=pl.ANY),
        ],
        out_specs=pl.BlockSpec(memory_space=pltpu.VMEM),
        scratch_shapes=[
            pltpu.VMEM((N_DEV * M_BLK, M_BLK), jnp.bfloat16),
            pltpu.VMEM((N_DEV, M_BLK, M_BLK), jnp.bfloat16),
            pltpu.VMEM((_NBUF, M_BLK, n), jnp.float32),
            pltpu.VMEM(
                (2, M_BLK, n) if _BF16W else (1, 8, 128), jnp.bfloat16
            ),
            pltpu.VMEM((N_DEV, 128), jnp.float32),
            pltpu.SemaphoreType.DMA((N_DEV - 1,)),
            pltpu.SemaphoreType.DMA((N_DEV - 1,)),
            pltpu.SemaphoreType.DMA((N_DEV - 1,)),
            pltpu.SemaphoreType.DMA((N_DEV - 1,)),
            pltpu.SemaphoreType.DMA((_NBUF, _WSPLIT)),
        ],
        compiler_params=pltpu.CompilerParams(
            collective_id=None if _NOCOMM else 0,
            vmem_limit_bytes=100 * 1024 * 1024,
        ),
    )(x, w_mat)


# device time: 65494 ns/iter; 1.3233x vs baseline; 1.0191x over previous
import os

import jax
import jax.numpy as jnp
from jax import lax
from jax.experimental import pallas as pl
from jax.experimental.pallas import tpu as pltpu

N_DEV = 16
M_BLK = 256

_FLAGS = set(filter(None, os.environ.get("KVAR", "").split(",")))
_NOCOMM = "nocomm" in _FLAGS
_NOQUANT = "noquant" in _FLAGS
_BF16W = "bf16w" in _FLAGS
_DMAONLY = "dmaonly" in _FLAGS
_NBUF = int(os.environ.get("NBUF", "2"))
_WSPLIT = int(os.environ.get("WSPLIT", "1"))

_DEV_ID_TYPE = getattr(pltpu, "DeviceIdType", None) or pl.DeviceIdType


def kernel(x, w_mat):
    m_full, k_shard = x.shape
    k_full, n = w_mat.shape
    assert m_full == N_DEV * M_BLK and k_shard == M_BLK

    def body(x_ref, w_ref, out_ref, xbf_ref, xg_ref, wbuf_ref, wbf_ref,
             amax_ref, ssem, rsem, assem, arsem, wsem):
        my = lax.axis_index("i")

        def j_of(d):
            return lax.rem(my + N_DEV - d, N_DEV)

        order = [0]
        for k in range(1, N_DEV // 2 + 1):
            order.append(k)
            if k != N_DEV - k:
                order.append(N_DEV - k)

        rows_per = M_BLK // _WSPLIT

        def w_dma(d, slot):
            j = j_of(d)
            descs = []
            for h in range(_WSPLIT):
                descs.append(pltpu.make_async_copy(
                    w_ref.at[pl.ds(j * M_BLK + h * rows_per, rows_per), :],
                    wbuf_ref.at[slot, pl.ds(h * rows_per, rows_per), :],
                    wsem.at[slot, h],
                ))
            return descs

        def start_all(descs):
            for c in descs:
                c.start()

        def wait_all(descs):
            for c in descs:
                c.wait()

        dmas = {}
        for idx in range(_NBUF):
            dmas[idx] = w_dma(order[idx], idx % _NBUF)
            start_all(dmas[idx])

        xbf_ref[...] = x_ref[...].astype(jnp.bfloat16)

        xg_ref[my, :, :] = xbf_ref[pl.ds(my * M_BLK, M_BLK), :]

        send_descs = []
        if not _NOCOMM:
            barrier_sem = pltpu.get_barrier_semaphore()
            for d in range(1, N_DEV):
                pl.semaphore_signal(
                    barrier_sem, inc=1,
                    device_id=(lax.rem(my + d, N_DEV),),
                    device_id_type=_DEV_ID_TYPE.MESH,
                )
            pl.semaphore_wait(barrier_sem, N_DEV - 1)
            for d in order[1:]:
                dst = lax.rem(my + d, N_DEV)
                rdma = pltpu.make_async_remote_copy(
                    src_ref=xbf_ref.at[pl.ds(dst * M_BLK, M_BLK), :],
                    dst_ref=xg_ref.at[my],
                    send_sem=ssem.at[d - 1],
                    recv_sem=rsem.at[d - 1],
                    device_id=(dst,),
                    device_id_type=_DEV_ID_TYPE.MESH,
                )
                rdma.start()
                send_descs.append(rdma)

        for idx in range(N_DEV):
            d = order[idx]
            if d > 0 and not _NOCOMM:
                recv = pltpu.make_async_remote_copy(
                    src_ref=xbf_ref.at[pl.ds(0, M_BLK), :],
                    dst_ref=xg_ref.at[j_of(d)],
                    send_sem=ssem.at[d - 1],
                    recv_sem=rsem.at[d - 1],
                    device_id=(my,),
                    device_id_type=_DEV_ID_TYPE.MESH,
                )
                recv.wait_recv()
            wait_all(dmas[idx])
            slot = idx % _NBUF
            if _DMAONLY:
                if idx + _NBUF < N_DEV:
                    dmas[idx + _NBUF] = w_dma(order[idx + _NBUF], slot)
                    start_all(dmas[idx + _NBUF])
                if idx == N_DEV - 1:
                    out_ref[...] = wbuf_ref[slot]
                continue
            if _BF16W:
                wbf_ref[idx % 2] = wbuf_ref[slot].astype(jnp.bfloat16)
                contrib = jnp.dot(
                    xg_ref[j_of(d)], wbf_ref[idx % 2],
                    preferred_element_type=jnp.float32,
                )
            else:
                contrib = jnp.dot(
                    xg_ref[j_of(d)].astype(jnp.float32), wbuf_ref[slot],
                    preferred_element_type=jnp.float32,
                )
            if idx + _NBUF < N_DEV:
                dmas[idx + _NBUF] = w_dma(order[idx + _NBUF], slot)
                start_all(dmas[idx + _NBUF])
            if idx == 0:
                out_ref[...] = contrib
            else:
                out_ref[...] += contrib

        for rdma in send_descs:
            rdma.wait_send()

        local_amax = jnp.max(jnp.abs(out_ref[...]))
        if not _NOCOMM:
            amax_ref[pl.ds(my, 1), :] = (
                jnp.zeros((1, 128), jnp.float32) + local_amax
            )
            a_sends = []
            for d in range(1, N_DEV):
                dst = lax.rem(my + d, N_DEV)
                r = pltpu.make_async_remote_copy(
                    src_ref=amax_ref.at[pl.ds(my, 1), :],
                    dst_ref=amax_ref.at[pl.ds(my, 1), :],
                    send_sem=assem.at[d - 1],
                    recv_sem=arsem.at[d - 1],
                    device_id=(dst,),
                    device_id_type=_DEV_ID_TYPE.MESH,
                )
                r.start()
                a_sends.append(r)
            for d in range(1, N_DEV):
                recv = pltpu.make_async_remote_copy(
                    src_ref=amax_ref.at[pl.ds(j_of(d), 1), :],
                    dst_ref=amax_ref.at[pl.ds(j_of(d), 1), :],
                    send_sem=assem.at[d - 1],
                    recv_sem=arsem.at[d - 1],
                    device_id=(my,),
                    device_id_type=_DEV_ID_TYPE.MESH,
                )
                recv.wait_recv()
            for r in a_sends:
                r.wait_send()
            g_amax = jnp.max(amax_ref[...])
        else:
            g_amax = local_amax

        if not _NOQUANT:
            inv_scale = 127.0 / g_amax
            scale = g_amax / 127.0
            q = jnp.clip(
                jnp.round(out_ref[...] * inv_scale), -127.0, 127.0
            )
            out_ref[...] = q * scale

    return pl.pallas_call(
        body,
        out_shape=jax.ShapeDtypeStruct((M_BLK, n), jnp.float32),
        in_specs=[
            pl.BlockSpec(memory_space=pltpu.VMEM),
            pl.BlockSpec(memory_space=pl.ANY),
        ],
        out_specs=pl.BlockSpec(memory_space=pltpu.VMEM),
        scratch_shapes=[
            pltpu.VMEM((N_DEV * M_BLK, M_BLK), jnp.bfloat16),
            pltpu.VMEM((N_DEV, M_BLK, M_BLK), jnp.bfloat16),
            pltpu.VMEM((_NBUF, M_BLK, n), jnp.float32),
            pltpu.VMEM(
                (2, M_BLK, n) if _BF16W else (1, 8, 128), jnp.bfloat16
            ),
            pltpu.VMEM((N_DEV, 128), jnp.float32),
            pltpu.SemaphoreType.DMA((N_DEV - 1,)),
            pltpu.SemaphoreType.DMA((N_DEV - 1,)),
            pltpu.SemaphoreType.DMA((N_DEV - 1,)),
            pltpu.SemaphoreType.DMA((N_DEV - 1,)),
            pltpu.SemaphoreType.DMA((_NBUF, _WSPLIT)),
        ],
        compiler_params=pltpu.CompilerParams(
            collective_id=None if _NOCOMM else 0,
            vmem_limit_bytes=100 * 1024 * 1024,
        ),
    )(x, w_mat)
